# baseline (speedup 1.0000x reference)
import functools

import jax
import jax.numpy as jnp
from jax import lax
from jax.experimental import pallas as pl
from jax.experimental.pallas import tpu as pltpu
from jax.experimental.pallas import tpu_sc as plsc

N = 10000
K = 16
C = 128
O = 128

NW = 32
NPT = 320
NPAD = NW * NPT
G = 8
EG = G * K
NGROUPS = NPT // G

BN = 512


def _tc_body(x_ref, dis_ref, A_ref, W2_ref, b_ref, attw_ref, attb_ref,
             y1_ref, y2_ref, supp_ref):
    xb = x_ref[...]
    A = A_ref[...]
    W2 = W2_ref[...]
    dn = (((0,), (1,)), ((), ()))
    y1 = lax.dot_general(xb, A, dn, preferred_element_type=jnp.float32)
    y1_ref[...] = y1 + b_ref[...]
    y2_ref[...] = lax.dot_general(xb, W2, dn, preferred_element_type=jnp.float32)
    disb = dis_ref[...]
    scal = jnp.sum(disb * attw_ref[...], axis=1, keepdims=True) + attb_ref[0, 0]
    scal = jnp.tanh(scal) + 1.0
    supp_ref[...] = 2.0 * jax.nn.sigmoid(-disb * scal)


_tc_call = pl.pallas_call(
    _tc_body,
    grid=(NPAD // BN,),
    in_specs=[
        pl.BlockSpec((C, BN), lambda i: (0, i)),
        pl.BlockSpec((BN, K), lambda i: (i, 0)),
        pl.BlockSpec((O, C), lambda i: (0, 0)),
        pl.BlockSpec((O, C), lambda i: (0, 0)),
        pl.BlockSpec((1, O), lambda i: (0, 0)),
        pl.BlockSpec((1, K), lambda i: (0, 0)),
        pl.BlockSpec((1, 1), lambda i: (0, 0)),
    ],
    out_specs=[
        pl.BlockSpec((BN, O), lambda i: (i, 0)),
        pl.BlockSpec((BN, O), lambda i: (i, 0)),
        pl.BlockSpec((BN, K), lambda i: (i, 0)),
    ],
    out_shape=[
        jax.ShapeDtypeStruct((NPAD, O), jnp.float32),
        jax.ShapeDtypeStruct((NPAD, O), jnp.float32),
        jax.ShapeDtypeStruct((NPAD, K), jnp.float32),
    ],
)


_sc_mesh = plsc.VectorSubcoreMesh(core_axis_name="c", subcore_axis_name="s")


@functools.partial(
    pl.kernel,
    out_type=jax.ShapeDtypeStruct((NPAD, O), jnp.float32),
    mesh=_sc_mesh,
    scratch_types=[
        pltpu.VMEM((NPT * K,), jnp.int32),
        pltpu.VMEM((NPT * K,), jnp.int32),
        pltpu.VMEM((NPT * K,), jnp.float32),
        pltpu.VMEM((2 * EG, O), jnp.float32),
        pltpu.VMEM((2 * EG, O), jnp.float32),
        pltpu.VMEM((NPT, O), jnp.float32),
        pltpu.SemaphoreType.DMA,
        pltpu.SemaphoreType.DMA,
    ],
)
def _sc_kernel(y1_hbm, y2_hbm, idxi_hbm, idxj_hbm, supp_hbm, out_hbm,
               idxi_v, idxj_v, supp_v, r1_v, r2_v, out_v, semA, semB):
    wid = lax.axis_index("s") * 2 + lax.axis_index("c")
    ebase = wid * (NPT * K)
    nbase = wid * NPT
    sems = [semA, semB]

    pltpu.sync_copy(idxi_hbm.at[pl.ds(ebase, NPT * K)], idxi_v)
    pltpu.sync_copy(idxj_hbm.at[pl.ds(ebase, NPT * K)], idxj_v)
    pltpu.sync_copy(supp_hbm.at[pl.ds(ebase, NPT * K)], supp_v)

    def fire(gi, p):
        pltpu.async_copy(y1_hbm.at[idxi_v.at[pl.ds(gi * EG, EG)]],
                         r1_v.at[pl.ds(p * EG, EG)], sems[p])
        pltpu.async_copy(y2_hbm.at[idxj_v.at[pl.ds(gi * EG, EG)]],
                         r2_v.at[pl.ds(p * EG, EG)], sems[p])

    def drain(gi, p):
        pltpu.make_async_copy(y1_hbm.at[idxi_v.at[pl.ds(gi * EG, EG)]],
                              r1_v.at[pl.ds(p * EG, EG)], sems[p]).wait()
        pltpu.make_async_copy(y2_hbm.at[idxj_v.at[pl.ds(gi * EG, EG)]],
                              r2_v.at[pl.ds(p * EG, EG)], sems[p]).wait()

    def compute(gi, p):
        def node(n, carry2):
            accs = [jnp.zeros((16,), jnp.float32) for _ in range(O // 16)]
            sv = supp_v[pl.ds(gi * EG + n * K, K)]
            for k in range(K):
                e = p * EG + n * K + k
                s = sv[k]
                for c8 in range(O // 16):
                    v = r1_v[e, pl.ds(c8 * 16, 16)] + r2_v[e, pl.ds(c8 * 16, 16)]
                    v = jnp.maximum(v, 0.0) * s
                    accs[c8] = jnp.maximum(accs[c8], v)
            for c8 in range(O // 16):
                out_v[gi * G + n, pl.ds(c8 * 16, 16)] = accs[c8]
            return carry2

        lax.fori_loop(0, G, node, 0)

    fire(0, 0)

    @pl.loop(0, NGROUPS, step=2)
    def body(g):
        for p in range(2):
            gi = g + p

            @pl.when(gi + 1 < NGROUPS)
            def _():
                fire(gi + 1, 1 - p)

            drain(gi, p)
            compute(gi, p)

    pltpu.sync_copy(out_v, out_hbm.at[pl.ds(nbase, NPT)])


def kernel(x, edge_index, pos, dis, W, b, att_W, att_b):
    del pos
    xf = x[0, :, :, 0]
    xf = jnp.pad(xf, ((0, 0), (0, NPAD - N)))
    disf = jnp.pad(dis[0], ((0, NPAD - N), (0, 0)))
    A = W[:, :C] - W[:, C:]
    W2 = W[:, C:]
    y1, y2, supp = _tc_call(
        xf, disf, A, W2, b.reshape(1, O).astype(jnp.float32),
        att_W.astype(jnp.float32), att_b.reshape(1, 1).astype(jnp.float32))
    idx = edge_index.astype(jnp.int32).reshape(2, N * K)
    idx = jnp.pad(idx, ((0, 0), (0, (NPAD - N) * K)))
    out = _sc_kernel(y1, y2, idx[1], idx[0], supp.reshape(-1))
    return out[:N].T.reshape(1, O, N, 1)


# speedup vs baseline: 1.0070x; 1.0070x over previous
import functools

import jax
import jax.numpy as jnp
from jax import lax
from jax.experimental import pallas as pl
from jax.experimental.pallas import tpu as pltpu
from jax.experimental.pallas import tpu_sc as plsc

N = 10000
K = 16
C = 128
O = 128

NW = 32
NPT = 320
NPAD = NW * NPT
G = 8
EG = G * K
NGROUPS = NPT // G

BN = 512


def _tc_body(x_ref, dis_ref, A_ref, W2_ref, b_ref, attw_ref, attb_ref,
             y1_ref, y2_ref, supp_ref):
    xb = x_ref[...]
    A = A_ref[...]
    W2 = W2_ref[...]
    dn = (((0,), (1,)), ((), ()))
    y1 = lax.dot_general(xb, A, dn, preferred_element_type=jnp.float32)
    y1_ref[...] = y1 + b_ref[...]
    y2_ref[...] = lax.dot_general(xb, W2, dn, preferred_element_type=jnp.float32)
    disb = dis_ref[...]
    scal = jnp.sum(disb * attw_ref[...], axis=1, keepdims=True) + attb_ref[0, 0]
    scal = jnp.tanh(scal) + 1.0
    supp_ref[...] = 2.0 * jax.nn.sigmoid(-disb * scal)


_tc_call = pl.pallas_call(
    _tc_body,
    grid=(NPAD // BN,),
    in_specs=[
        pl.BlockSpec((C, BN), lambda i: (0, i)),
        pl.BlockSpec((BN, K), lambda i: (i, 0)),
        pl.BlockSpec((O, C), lambda i: (0, 0)),
        pl.BlockSpec((O, C), lambda i: (0, 0)),
        pl.BlockSpec((1, O), lambda i: (0, 0)),
        pl.BlockSpec((1, K), lambda i: (0, 0)),
        pl.BlockSpec((1, 1), lambda i: (0, 0)),
    ],
    out_specs=[
        pl.BlockSpec((BN, O), lambda i: (i, 0)),
        pl.BlockSpec((BN, O), lambda i: (i, 0)),
        pl.BlockSpec((BN, K), lambda i: (i, 0)),
    ],
    out_shape=[
        jax.ShapeDtypeStruct((NPAD, O), jnp.float32),
        jax.ShapeDtypeStruct((NPAD, O), jnp.float32),
        jax.ShapeDtypeStruct((NPAD, K), jnp.float32),
    ],
)


_sc_mesh = plsc.VectorSubcoreMesh(core_axis_name="c", subcore_axis_name="s")


@functools.partial(
    pl.kernel,
    out_type=jax.ShapeDtypeStruct((NPAD, O), jnp.float32),
    mesh=_sc_mesh,
    scratch_types=[
        pltpu.VMEM((NPT * K,), jnp.int32),
        pltpu.VMEM((NPT * K,), jnp.int32),
        pltpu.VMEM((NPT * K,), jnp.float32),
        pltpu.VMEM((2 * EG, O), jnp.float32),
        pltpu.VMEM((2 * EG, O), jnp.float32),
        pltpu.VMEM((NPT, O), jnp.float32),
        pltpu.SemaphoreType.DMA,
        pltpu.SemaphoreType.DMA,
        pltpu.SemaphoreType.DMA,
        pltpu.SemaphoreType.DMA,
        pltpu.SemaphoreType.DMA,
        pltpu.SemaphoreType.DMA,
        pltpu.SemaphoreType.DMA,
        pltpu.SemaphoreType.DMA,
    ],
)
def _sc_kernel(y1_hbm, y2_hbm, idxi_hbm, idxj_hbm, supp_hbm, out_hbm,
               idxi_v, idxj_v, supp_v, r1_v, r2_v, out_v,
               s0, s1, s2, s3, s4, s5, s6, s7):
    wid = lax.axis_index("s") * 2 + lax.axis_index("c")
    ebase = wid * (NPT * K)
    nbase = wid * NPT
    sems = [[s0, s1, s2, s3], [s4, s5, s6, s7]]
    EH = EG // 2

    pltpu.sync_copy(idxi_hbm.at[pl.ds(ebase, NPT * K)], idxi_v)
    pltpu.sync_copy(idxj_hbm.at[pl.ds(ebase, NPT * K)], idxj_v)
    pltpu.sync_copy(supp_hbm.at[pl.ds(ebase, NPT * K)], supp_v)

    def _descs(gi, p):
        out = []
        for h in range(2):
            out.append((y1_hbm.at[idxi_v.at[pl.ds(gi * EG + h * EH, EH)]],
                        r1_v.at[pl.ds(p * EG + h * EH, EH)], sems[p][h]))
            out.append((y2_hbm.at[idxj_v.at[pl.ds(gi * EG + h * EH, EH)]],
                        r2_v.at[pl.ds(p * EG + h * EH, EH)], sems[p][2 + h]))
        return out

    def fire(gi, p):
        for src, dst, sem in _descs(gi, p):
            pltpu.async_copy(src, dst, sem)

    def drain(gi, p):
        for src, dst, sem in _descs(gi, p):
            pltpu.make_async_copy(src, dst, sem).wait()

    def compute(gi, p):
        def node(n, carry2):
            accs = [jnp.zeros((16,), jnp.float32) for _ in range(O // 16)]
            sv = supp_v[pl.ds(gi * EG + n * K, K)]
            for k in range(K):
                e = p * EG + n * K + k
                s = sv[k]
                for c8 in range(O // 16):
                    v = r1_v[e, pl.ds(c8 * 16, 16)] + r2_v[e, pl.ds(c8 * 16, 16)]
                    v = jnp.maximum(v, 0.0) * s
                    accs[c8] = jnp.maximum(accs[c8], v)
            for c8 in range(O // 16):
                out_v[gi * G + n, pl.ds(c8 * 16, 16)] = accs[c8]
            return carry2

        lax.fori_loop(0, G, node, 0)

    fire(0, 0)

    @pl.loop(0, NGROUPS, step=2)
    def body(g):
        for p in range(2):
            gi = g + p

            @pl.when(gi + 1 < NGROUPS)
            def _():
                fire(gi + 1, 1 - p)

            drain(gi, p)
            compute(gi, p)

    pltpu.sync_copy(out_v, out_hbm.at[pl.ds(nbase, NPT)])


def kernel(x, edge_index, pos, dis, W, b, att_W, att_b):
    del pos
    xf = x[0, :, :, 0]
    xf = jnp.pad(xf, ((0, 0), (0, NPAD - N)))
    disf = jnp.pad(dis[0], ((0, NPAD - N), (0, 0)))
    A = W[:, :C] - W[:, C:]
    W2 = W[:, C:]
    y1, y2, supp = _tc_call(
        xf, disf, A, W2, b.reshape(1, O).astype(jnp.float32),
        att_W.astype(jnp.float32), att_b.reshape(1, 1).astype(jnp.float32))
    idx = edge_index.astype(jnp.int32).reshape(2, N * K)
    idx = jnp.pad(idx, ((0, 0), (0, (NPAD - N) * K)))
    out = _sc_kernel(y1, y2, idx[1], idx[0], supp.reshape(-1))
    return out[:N].T.reshape(1, O, N, 1)
